# Initial kernel scaffold; baseline (speedup 1.0000x reference)
import jax
import jax.numpy as jnp
from jax.experimental import pallas as pl


def kernel(x, y, sc, Q, gamma, W_gcn, b_gcn, gn_weight, gn_bias):
    raise NotImplementedError("write your pallas kernel here")


# speedup vs baseline: 12.8492x; 12.8492x over previous
import jax
import jax.numpy as jnp
from jax import lax
from jax.experimental import pallas as pl
from jax.experimental.pallas import tpu as pltpu
from jax.experimental.pallas import tpu_sc as plsc

N = 10000
C = 128
E = 320000
NOSC = 4
EPS_SPHERE = 1e-6
EPS_GN = 1e-5

NC = 2
NS = 16
NW = NC * NS
EPW = E // NW
CH = 80
NCHUNK = EPW // CH
NP = 10240
NPS = NP // NS
ZR = 128
DEGW = 16

_F32 = jnp.float32


def _agg_body(hs_hbm, src_hbm, dst_hbm, zer_hbm, out_hbm,
              idx_s, idx_d, rows_v, acc_sh, sem):
    cid = lax.axis_index("c")
    sid = lax.axis_index("s")
    wid = cid * NS + sid

    @pl.when(sid == 0)
    def _():
        pltpu.sync_copy(zer_hbm, acc_sh)

    plsc.subcore_barrier()

    def body(g, carry):
        off = pl.multiple_of(wid * EPW + g * CH, 8)
        pltpu.sync_copy(src_hbm.at[pl.ds(off, CH)], idx_s)
        pltpu.sync_copy(dst_hbm.at[pl.ds(off, CH)], idx_d)
        pltpu.async_copy(hs_hbm.at[idx_s], rows_v, sem).wait()
        pltpu.sync_copy(rows_v, acc_sh.at[idx_d], add=True)
        return carry

    lax.fori_loop(0, NCHUNK, body, 0)
    plsc.subcore_barrier()

    @pl.when(sid == 0)
    def _():
        pltpu.sync_copy(acc_sh, out_hbm.at[cid])


_agg_call = pl.kernel(
    _agg_body,
    out_type=jax.ShapeDtypeStruct((NC, NP, C), _F32),
    mesh=plsc.VectorSubcoreMesh(core_axis_name="c", subcore_axis_name="s"),
    scratch_types=[
        pltpu.VMEM((CH,), jnp.int32),
        pltpu.VMEM((CH,), jnp.int32),
        pltpu.VMEM((CH, C), _F32),
        pltpu.VMEM_SHARED((NP, C), _F32),
        pltpu.SemaphoreType.DMA,
    ],
)


def _gmat():
    ii = lax.broadcasted_iota(jnp.int32, (C, C), 0) // NOSC
    jj = lax.broadcasted_iota(jnp.int32, (C, C), 1) // NOSC
    return (ii == jj).astype(_F32)


def _gdot(p, G):
    return jnp.dot(p, G, precision=lax.Precision.HIGHEST,
                   preferred_element_type=_F32)


def _sphere(v, G):
    n2 = jnp.clip(_gdot(v * v, G), EPS_SPHERE, None)
    return v * lax.rsqrt(n2)


BNP = 2048


def _stats_body(y_ref, colsum_ref, colsq_ref):
    i = pl.program_id(0)
    y = y_ref[...]
    s1 = jnp.sum(y, axis=0, keepdims=True)
    s2 = jnp.sum(y * y, axis=0, keepdims=True)

    @pl.when(i == 0)
    def _():
        colsum_ref[...] = s1
        colsq_ref[...] = s2

    @pl.when(i != 0)
    def _():
        colsum_ref[...] += s1
        colsq_ref[...] += s2


_stats_call = pl.pallas_call(
    _stats_body,
    grid=(NP // BNP,),
    in_specs=[pl.BlockSpec((BNP, C), lambda i: (i, 0))],
    out_specs=[pl.BlockSpec((1, C), lambda i: (0, 0)),
               pl.BlockSpec((1, C), lambda i: (0, 0))],
    out_shape=[jax.ShapeDtypeStruct((1, C), _F32),
               jax.ShapeDtypeStruct((1, C), _F32)],
)


def _pre_body(y_ref, x_ref, degp_ref, colsum_ref, colsq_ref, gnw_ref, gnb_ref,
              w_ref, y2_ref, xs0_ref, hs0_ref, disc_ref):
    G = _gmat()
    cnt = _F32(NOSC * N)
    mean = _gdot(colsum_ref[...], G) / cnt
    var = _gdot(colsq_ref[...], G) / cnt - mean * mean
    yn = (y_ref[...] - mean) * lax.rsqrt(var + EPS_GN)
    yv = yn * gnw_ref[...] + gnb_ref[...]
    y2_ref[...] = _sphere(yv, G)

    xs0 = _sphere(x_ref[...], G)
    xs0_ref[...] = xs0

    deg = degp_ref[0][:, 0:1] + degp_ref[1][:, 0:1] + 1.0
    disc = jnp.broadcast_to(lax.rsqrt(deg), (BNP, C))
    disc_ref[...] = disc
    hs0_ref[...] = jnp.dot(xs0, w_ref[...], preferred_element_type=_F32) * disc


_prow_spec = pl.BlockSpec((BNP, C), lambda i: (i, 0))
_pre_call = pl.pallas_call(
    _pre_body,
    grid=(NP // BNP,),
    in_specs=[
        _prow_spec,
        _prow_spec,
        pl.BlockSpec((NC, BNP, C), lambda i: (0, i, 0)),
        pl.BlockSpec((1, C), lambda i: (0, 0)),
        pl.BlockSpec((1, C), lambda i: (0, 0)),
        pl.BlockSpec((1, C), lambda i: (0, 0)),
        pl.BlockSpec((1, C), lambda i: (0, 0)),
        pl.BlockSpec((C, C), lambda i: (0, 0)),
    ],
    out_specs=[_prow_spec, _prow_spec, _prow_spec, _prow_spec],
    out_shape=[
        jax.ShapeDtypeStruct((NP, C), _F32),
        jax.ShapeDtypeStruct((NP, C), _F32),
        jax.ShapeDtypeStruct((NP, C), _F32),
        jax.ShapeDtypeStruct((NP, C), _F32),
    ],
)


BN = 2048


def _step_body(xs_ref, aggp_ref, hs_ref, disc_ref, y2_ref, w_ref, b_ref,
               gam_ref, xsn_ref, hsn_ref):
    G = _gmat()
    xs = xs_ref[...]
    dis = disc_ref[...]
    c = dis * (aggp_ref[0] + aggp_ref[1] + hs_ref[...]) + b_ref[...] + y2_ref[...]
    sim = _gdot(xs * c, G)
    dxdt = c - sim * xs
    xn = xs + gam_ref[...] * dxdt
    xsn = _sphere(xn, G)
    xsn_ref[...] = xsn
    hsn_ref[...] = jnp.dot(xsn, w_ref[...], preferred_element_type=_F32) * dis


_row_spec = pl.BlockSpec((BN, C), lambda i: (i, 0))
_step_call = pl.pallas_call(
    _step_body,
    grid=(NP // BN,),
    in_specs=[
        _row_spec,
        pl.BlockSpec((NC, BN, C), lambda i: (0, i, 0)),
        _row_spec,
        _row_spec,
        _row_spec,
        pl.BlockSpec((C, C), lambda i: (0, 0)),
        pl.BlockSpec((1, C), lambda i: (0, 0)),
        pl.BlockSpec((1, 1), lambda i: (0, 0)),
    ],
    out_specs=[_row_spec, _row_spec],
    out_shape=[
        jax.ShapeDtypeStruct((NP, C), _F32),
        jax.ShapeDtypeStruct((NP, C), _F32),
    ],
)


def kernel(x, y, sc, Q, gamma, W_gcn, b_gcn, gn_weight, gn_bias):
    pad = jnp.zeros((NP - N, C), _F32)
    x2 = jnp.concatenate([x.reshape(N, C), pad])
    y2in = jnp.concatenate([y.reshape(N, C), pad])
    src = sc[0]
    dst = sc[1]
    ones_tab = jnp.ones((NP, C), _F32)
    zer_agg = jnp.zeros((NP, C), _F32)
    gnw = gn_weight.reshape(1, C)
    gnb = gn_bias.reshape(1, C)
    bb = b_gcn.reshape(1, C)
    gam = jnp.asarray(gamma, _F32).reshape(1, 1)

    degp = _agg_call(ones_tab, src, dst, zer_agg)
    colsum, colsq = _stats_call(y2in)
    y2n, xs0, hs0, disc = _pre_call(y2in, x2, degp, colsum, colsq, gnw, gnb, W_gcn)

    def body(_, carry):
        xs, hs = carry
        aggp = _agg_call(hs, src, dst, zer_agg)
        xsn, hsn = _step_call(xs, aggp, hs, disc, y2n, W_gcn, bb, gam)
        return (xsn, hsn)

    xs, _ = lax.fori_loop(0, Q, body, (xs0, hs0))
    return xs[:N].reshape(1, N, C)
